# baseline (speedup 1.0000x reference)
import functools

import jax
import jax.numpy as jnp
from jax import lax
from jax.experimental import pallas as pl
from jax.experimental.pallas import tpu as pltpu
from jax.experimental.pallas import tpu_sc as plsc

N_T = 50000
N_U = 10000
N_M = 5000
D = 128
E = 200000

ER = 1792
EP = ER * 128
BUCK = 6256
AGG_R = 8 * BUCK
ACC_R = 6400
DUMP = BUCK + 8
DPAD = 50112
FLUSH = 112

LU = 10240
LM = 6144
LT = 51200

_mesh = plsc.VectorSubcoreMesh(core_axis_name="c", subcore_axis_name="s")

_GDN = lax.GatherDimensionNumbers(offset_dims=(), collapsed_slice_dims=(0,),
                                  start_index_map=(0,))


def _take16(vec, idx):
    return lax.gather(vec, idx[:, None], _GDN, (1,),
                      mode=lax.GatherScatterMode.PROMISE_IN_BOUNDS)


def _pad_edges(a, fill):
    pad = jnp.full((EP - E,), fill, jnp.int32)
    return jnp.concatenate([a.astype(jnp.int32), pad]).reshape(ER, 128)


def _hist_body(s_ut, d_ut, s_mt, d_mt, s_tt, d_tt,
               o_ut, o_mt, o_tt, i_ut, i_mt, i_tt,
               stage, ones_v, zbuf,
               h_ou, h_om, h_ot, h_iu, h_im, h_it, hsem):
    core = lax.axis_index("c")
    sub = lax.axis_index("s")

    def fill(ref, n, val):
        v = jnp.full((16,), val, jnp.float32)

        def body(i, _):
            ref[pl.ds(i * 16, 16)] = v
            return 0
        lax.fori_loop(0, n // 16, body, 0)

    fill(zbuf, LT // 16, 0.0)
    fill(ones_v, 128, 1.0)

    pairs = [(s_ut, h_ou, o_ut, LU), (s_mt, h_om, o_mt, LM),
             (s_tt, h_ot, o_tt, LT), (d_ut, h_iu, i_ut, LT),
             (d_mt, h_im, i_mt, LT), (d_tt, h_it, i_tt, LT)]

    for _, hist, _, L in pairs:
        share = L // 16
        pltpu.sync_copy(zbuf.at[pl.ds(0, share)],
                        hist.at[pl.ds(sub * share, share)])
    plsc.subcore_barrier()

    row0 = core * (ER // 2) + sub * 56
    pltpu.sync_copy(pairs[0][0].at[pl.ds(row0, 56), :], stage.at[0])
    for k, (arr, hist, _, L) in enumerate(pairs):
        sl = k & 1
        for g in range(56):
            pltpu.async_copy(ones_v, hist.at[stage.at[sl, g]], hsem,
                             add=True)
        if k + 1 < len(pairs):
            pltpu.sync_copy(pairs[k + 1][0].at[pl.ds(row0, 56), :],
                            stage.at[1 - sl])
        for g in range(56):
            pltpu.make_async_copy(ones_v, hist.at[stage.at[sl, g]],
                                  hsem).wait()
    plsc.subcore_barrier()

    for _, hist, out, L in pairs:
        share = L // 16
        pltpu.sync_copy(hist.at[pl.ds(sub * share, share)],
                        out.at[pl.ds(core * L + sub * share, share)])


@functools.partial(
    pl.kernel, mesh=_mesh,
    compiler_params=pltpu.CompilerParams(needs_layout_passes=False),
    out_type=[jax.ShapeDtypeStruct((2 * LU,), jnp.float32),
              jax.ShapeDtypeStruct((2 * LM,), jnp.float32),
              jax.ShapeDtypeStruct((2 * LT,), jnp.float32),
              jax.ShapeDtypeStruct((2 * LT,), jnp.float32),
              jax.ShapeDtypeStruct((2 * LT,), jnp.float32),
              jax.ShapeDtypeStruct((2 * LT,), jnp.float32)],
    scratch_types=[pltpu.VMEM((2, 56, 128), jnp.int32),
                   pltpu.VMEM((128,), jnp.float32),
                   pltpu.VMEM((LT // 16,), jnp.float32),
                   pltpu.VMEM_SHARED((LU,), jnp.float32),
                   pltpu.VMEM_SHARED((LM,), jnp.float32),
                   pltpu.VMEM_SHARED((LT,), jnp.float32),
                   pltpu.VMEM_SHARED((LT,), jnp.float32),
                   pltpu.VMEM_SHARED((LT,), jnp.float32),
                   pltpu.VMEM_SHARED((LT,), jnp.float32),
                   pltpu.SemaphoreType.DMA])
def _hist_kernel(*args):
    _hist_body(*args)


def _norm_body(t_ref, d_ref, o_ref):
    d = d_ref[0, :] + d_ref[1, :]
    o_ref[...] = t_ref[...] * lax.rsqrt(jnp.maximum(d, 1.0))[:, None]


def _norm(table, deg, n):
    blk = 512
    nb = pl.cdiv(n, blk)
    return pl.pallas_call(
        _norm_body,
        grid=(nb,),
        in_specs=[pl.BlockSpec((blk, D), lambda i: (i, 0)),
                  pl.BlockSpec((2, blk), lambda i: (0, i))],
        out_specs=pl.BlockSpec((blk, D), lambda i: (i, 0)),
        out_shape=jax.ShapeDtypeStruct((n, D), jnp.float32),
    )(table, deg)


def _conv_body(h_ut, h_mt, h_tt, e_su, e_du, e_sm, e_dm, e_st, e_dt,
               a_ut, a_mt, a_tt,
               sstage, dstage, pbig, rows4, sidx4, didx4, acc,
               gs0, gs1, gs2, ss0, ss1, ss2):
    core = lax.axis_index("c")
    sub = lax.axis_index("s")
    iota = lax.iota(jnp.int32, 16)
    gsems = [gs0, gs1, gs2]
    ssems = [ss0, ss1, ss2]
    NR = 3

    def zero_slot0():
        z = jnp.zeros((16,), jnp.float32)

        def body(i, _):
            for k in range(8):
                rows4[0, i, pl.ds(k * 16, 16)] = z
            return 0
        lax.fori_loop(0, 128, body, 0)

    for h_tab, e_src, e_dst, agg in ((h_ut, e_su, e_du, a_ut),
                                     (h_mt, e_sm, e_dm, a_mt),
                                     (h_tt, e_st, e_dt, a_tt)):

        def bucket_body(b, _, h_tab=h_tab, e_src=e_src, e_dst=e_dst,
                        agg=agg):
            lo = core * (4 * BUCK) + b * BUCK
            zero_slot0()
            for j in range(3):
                pltpu.sync_copy(rows4.at[0],
                                acc.at[pl.ds(sub * 400 + j * 128, 128), :])
            pltpu.sync_copy(rows4.at[0, pl.ds(0, 16), :],
                            acc.at[pl.ds(sub * 400 + 384, 16), :])
            plsc.subcore_barrier()

            lane15 = jnp.full((16,), 15, jnp.int32)

            def chunk_pass(c, cntv):
                pltpu.sync_copy(
                    e_src.at[pl.ds(sub * 112 + c * 56, 56), :], sstage)
                pltpu.sync_copy(
                    e_dst.at[pl.ds(sub * 112 + c * 56, 56), :], dstage)

                def row_body(r, cntv):
                    for k in range(8):
                        s16 = sstage[r, pl.ds(k * 16, 16)]
                        d16 = dstage[r, pl.ds(k * 16, 16)]
                        m = (d16 >= lo) & (d16 < lo + BUCK)
                        mi = jnp.where(m, 1, 0)
                        pos = cntv + plsc.cumsum(mi) - 1
                        pk = s16 | ((d16 - lo) << 17)
                        plsc.store_scatter(pbig, [pos >> 7, pos & 127], pk,
                                           mask=m)
                        cntv = _take16(pos, lane15) + 1
                    return cntv
                return lax.fori_loop(0, 56, row_body, cntv)

            cntv = lax.fori_loop(0, 2, chunk_pass,
                                 jnp.zeros((16,), jnp.int32))
            didx4[0, pl.ds(0, 16)] = cntv
            cnt = didx4[0, pl.ds(0, 16)][0]
            nb = (cnt + 127) >> 7
            hi = nb << 7
            pdump = jnp.full((16,), DUMP << 17, jnp.int32)
            for k in range(8):
                p = cnt + k * 16 + iota
                m = p < hi
                plsc.store_scatter(pbig, [p >> 7, p & 127], pdump, mask=m)

            def super_body(sup, _):
                base = sup * NR
                for q in range(NR):
                    j = base + q

                    @pl.when((j < nb) & (j >= NR))
                    def _():
                        pltpu.make_async_copy(rows4.at[q],
                                              acc.at[didx4.at[q]],
                                              ssems[q]).wait()

                    @pl.when(j < nb)
                    def _():
                        for k in range(8):
                            v = pbig[j, pl.ds(k * 16, 16)]
                            sidx4[q, pl.ds(k * 16, 16)] = v & 131071
                            didx4[q, pl.ds(k * 16, 16)] = v >> 17
                        pltpu.async_copy(h_tab.at[sidx4.at[q]], rows4.at[q],
                                         gsems[q])
                for q in range(NR):
                    j = base + q

                    @pl.when(j < nb)
                    def _():
                        pltpu.make_async_copy(h_tab.at[sidx4.at[q]],
                                              rows4.at[q], gsems[q]).wait()
                        pltpu.async_copy(rows4.at[q], acc.at[didx4.at[q]],
                                         ssems[q], add=True)
                return 0

            nsup = (nb + NR - 1) // NR
            lax.fori_loop(0, nsup, super_body, 0)
            for q in range(NR):

                @pl.when(q < nb)
                def _():
                    pltpu.make_async_copy(rows4.at[q], acc.at[didx4.at[q]],
                                          ssems[q]).wait()
            plsc.subcore_barrier()

            @pl.when(sub < 15)
            def _():
                pltpu.sync_copy(acc.at[pl.ds(sub * 392, 392), :],
                                agg.at[pl.ds(lo + sub * 392, 392), :])

            @pl.when(sub == 15)
            def _():
                pltpu.sync_copy(acc.at[pl.ds(5880, 376), :],
                                agg.at[pl.ds(lo + 5880, 376), :])
            plsc.subcore_barrier()
            return 0

        lax.fori_loop(0, 4, bucket_body, 0)


@functools.partial(
    pl.kernel, mesh=_mesh,
    compiler_params=pltpu.CompilerParams(needs_layout_passes=False),
    out_type=[jax.ShapeDtypeStruct((AGG_R, D), jnp.float32),
              jax.ShapeDtypeStruct((AGG_R, D), jnp.float32),
              jax.ShapeDtypeStruct((AGG_R, D), jnp.float32)],
    scratch_types=[pltpu.VMEM((56, 128), jnp.int32),
                   pltpu.VMEM((56, 128), jnp.int32),
                   pltpu.VMEM((112, 128), jnp.int32),
                   pltpu.VMEM((3, 128, 128), jnp.float32),
                   pltpu.VMEM((3, 128), jnp.int32),
                   pltpu.VMEM((3, 128), jnp.int32),
                   pltpu.VMEM_SHARED((ACC_R, 128), jnp.float32),
                   pltpu.SemaphoreType.DMA, pltpu.SemaphoreType.DMA,
                   pltpu.SemaphoreType.DMA, pltpu.SemaphoreType.DMA,
                   pltpu.SemaphoreType.DMA, pltpu.SemaphoreType.DMA])
def _conv_kernel(*args):
    _conv_body(*args)


def _head_body(ag_ut, ag_mt, ag_tt, id_ut, id_mt, id_tt,
               w_ut, w_mt, w_tt, bsum, w_out, b_out,
               emb_ref, logits_ref):
    def rs(d_ref):
        d = d_ref[0, :] + d_ref[1, :]
        return lax.rsqrt(jnp.maximum(d, 1.0))[:, None]

    e = jnp.dot(ag_ut[...] * rs(id_ut), w_ut[...],
                preferred_element_type=jnp.float32)
    e += jnp.dot(ag_mt[...] * rs(id_mt), w_mt[...],
                 preferred_element_type=jnp.float32)
    e += jnp.dot(ag_tt[...] * rs(id_tt), w_tt[...],
                 preferred_element_type=jnp.float32)
    e += bsum[...]
    emb_ref[...] = e
    logits_ref[...] = jnp.dot(e, w_out[...],
                              preferred_element_type=jnp.float32) + b_out[...]


def _head(ag_ut, ag_mt, ag_tt, in_ut, in_mt, in_tt,
          W_ut, W_mt, W_tt, bsum, W_out, b_out):
    blk = 512
    nb = pl.cdiv(N_T, blk)
    row = pl.BlockSpec((blk, D), lambda i: (i, 0))
    col = pl.BlockSpec((2, blk), lambda i: (0, i))
    full = pl.BlockSpec((D, D), lambda i: (0, 0))
    return pl.pallas_call(
        _head_body,
        grid=(nb,),
        in_specs=[row, row, row, col, col, col, full, full, full,
                  pl.BlockSpec((1, D), lambda i: (0, 0)),
                  pl.BlockSpec((D, 1), lambda i: (0, 0)),
                  pl.BlockSpec((1, 1), lambda i: (0, 0))],
        out_specs=[row, pl.BlockSpec((blk, 1), lambda i: (i, 0))],
        out_shape=[jax.ShapeDtypeStruct((N_T, D), jnp.float32),
                   jax.ShapeDtypeStruct((N_T, 1), jnp.float32)],
    )(ag_ut, ag_mt, ag_tt, in_ut, in_mt, in_tt,
      W_ut, W_mt, W_tt, bsum, W_out, b_out)


def kernel(features, user_ids, merchant_ids,
           src_ut, dst_ut, src_mt, dst_mt, src_tt, dst_tt,
           emb_user, emb_merchant,
           W_ut, b_ut, W_mt, b_mt, W_tt, b_tt, W_out, b_out):
    h_user = jnp.take(emb_user, user_ids, axis=0)
    h_merchant = jnp.take(emb_merchant, merchant_ids, axis=0)

    e_su = _pad_edges(src_ut, N_U)
    e_du = _pad_edges(dst_ut, DPAD)
    e_sm = _pad_edges(src_mt, N_M)
    e_dm = _pad_edges(dst_mt, DPAD)
    e_st = _pad_edges(src_tt, N_T)
    e_dt = _pad_edges(dst_tt, DPAD)

    o_ut, o_mt, o_tt, i_ut, i_mt, i_tt = _hist_kernel(
        e_su, e_du, e_sm, e_dm, e_st, e_dt)

    hn_ut = _norm(h_user, o_ut.reshape(2, LU), N_U)
    hn_mt = _norm(h_merchant, o_mt.reshape(2, LM), N_M)
    hn_tt = _norm(features, o_tt.reshape(2, LT), N_T)

    ag_ut, ag_mt, ag_tt = _conv_kernel(
        hn_ut, hn_mt, hn_tt, e_su, e_du, e_sm, e_dm, e_st, e_dt)

    bsum = (b_ut + b_mt + b_tt).reshape(1, D)
    emb, logits = _head(ag_ut[:N_T], ag_mt[:N_T], ag_tt[:N_T],
                        i_ut.reshape(2, LT), i_mt.reshape(2, LT),
                        i_tt.reshape(2, LT),
                        W_ut, W_mt, W_tt, bsum, W_out, b_out.reshape(1, 1))
    return (logits, emb)


# speedup vs baseline: 1.3197x; 1.3197x over previous
import functools

import jax
import jax.numpy as jnp
from jax import lax
from jax.experimental import pallas as pl
from jax.experimental.pallas import tpu as pltpu
from jax.experimental.pallas import tpu_sc as plsc

N_T = 50000
N_U = 10000
N_M = 5000
D = 128
E = 200000

ER = 1792
EP = ER * 128
BUCK = 6256
AGG_R = 8 * BUCK
ACC_R = 6400
DUMP = BUCK + 8
DPAD = 50112
FLUSH = 112

LU = 10240
LM = 6144
LT = 51200

_mesh = plsc.VectorSubcoreMesh(core_axis_name="c", subcore_axis_name="s")

_GDN = lax.GatherDimensionNumbers(offset_dims=(), collapsed_slice_dims=(0,),
                                  start_index_map=(0,))


def _take16(vec, idx):
    return lax.gather(vec, idx[:, None], _GDN, (1,),
                      mode=lax.GatherScatterMode.PROMISE_IN_BOUNDS)


def _pad_edges(a, fill):
    pad = jnp.full((EP - E,), fill, jnp.int32)
    return jnp.concatenate([a.astype(jnp.int32), pad]).reshape(ER, 128)


def _hist_body(s_ut, d_ut, s_mt, d_mt, s_tt, d_tt,
               o_ut, o_mt, o_tt, i_ut, i_mt, i_tt,
               stage, ones_v, zbuf,
               h_ou, h_om, h_ot, h_iu, h_im, h_it, hsem):
    core = lax.axis_index("c")
    sub = lax.axis_index("s")

    def fill(ref, n, val):
        v = jnp.full((16,), val, jnp.float32)

        def body(i, _):
            ref[pl.ds(i * 16, 16)] = v
            return 0
        lax.fori_loop(0, n // 16, body, 0)

    fill(zbuf, LT // 16, 0.0)
    fill(ones_v, 128, 1.0)

    pairs = [(s_ut, h_ou, o_ut, LU), (s_mt, h_om, o_mt, LM),
             (s_tt, h_ot, o_tt, LT), (d_ut, h_iu, i_ut, LT),
             (d_mt, h_im, i_mt, LT), (d_tt, h_it, i_tt, LT)]

    for _, hist, _, L in pairs:
        share = L // 16
        pltpu.sync_copy(zbuf.at[pl.ds(0, share)],
                        hist.at[pl.ds(sub * share, share)])
    plsc.subcore_barrier()

    row0 = core * (ER // 2) + sub * 56
    pltpu.sync_copy(pairs[0][0].at[pl.ds(row0, 56), :], stage.at[0])
    for k, (arr, hist, _, L) in enumerate(pairs):
        sl = k & 1
        for g in range(56):
            pltpu.async_copy(ones_v, hist.at[stage.at[sl, g]], hsem,
                             add=True)
        if k + 1 < len(pairs):
            pltpu.sync_copy(pairs[k + 1][0].at[pl.ds(row0, 56), :],
                            stage.at[1 - sl])
        for g in range(56):
            pltpu.make_async_copy(ones_v, hist.at[stage.at[sl, g]],
                                  hsem).wait()
    plsc.subcore_barrier()

    for _, hist, out, L in pairs:
        share = L // 16
        pltpu.sync_copy(hist.at[pl.ds(sub * share, share)],
                        out.at[pl.ds(core * L + sub * share, share)])


@functools.partial(
    pl.kernel, mesh=_mesh,
    compiler_params=pltpu.CompilerParams(needs_layout_passes=False),
    out_type=[jax.ShapeDtypeStruct((2 * LU,), jnp.float32),
              jax.ShapeDtypeStruct((2 * LM,), jnp.float32),
              jax.ShapeDtypeStruct((2 * LT,), jnp.float32),
              jax.ShapeDtypeStruct((2 * LT,), jnp.float32),
              jax.ShapeDtypeStruct((2 * LT,), jnp.float32),
              jax.ShapeDtypeStruct((2 * LT,), jnp.float32)],
    scratch_types=[pltpu.VMEM((2, 56, 128), jnp.int32),
                   pltpu.VMEM((128,), jnp.float32),
                   pltpu.VMEM((LT // 16,), jnp.float32),
                   pltpu.VMEM_SHARED((LU,), jnp.float32),
                   pltpu.VMEM_SHARED((LM,), jnp.float32),
                   pltpu.VMEM_SHARED((LT,), jnp.float32),
                   pltpu.VMEM_SHARED((LT,), jnp.float32),
                   pltpu.VMEM_SHARED((LT,), jnp.float32),
                   pltpu.VMEM_SHARED((LT,), jnp.float32),
                   pltpu.SemaphoreType.DMA])
def _hist_kernel(*args):
    _hist_body(*args)


def _norm_body(t_ref, d_ref, o_ref):
    d = d_ref[0, :] + d_ref[1, :]
    o_ref[...] = t_ref[...] * lax.rsqrt(jnp.maximum(d, 1.0))[:, None]


def _norm(table, deg, n):
    blk = 512
    nb = pl.cdiv(n, blk)
    return pl.pallas_call(
        _norm_body,
        grid=(nb,),
        in_specs=[pl.BlockSpec((blk, D), lambda i: (i, 0)),
                  pl.BlockSpec((2, blk), lambda i: (0, i))],
        out_specs=pl.BlockSpec((blk, D), lambda i: (i, 0)),
        out_shape=jax.ShapeDtypeStruct((n, D), jnp.float32),
    )(table, deg)


def _conv_body(h_ut, h_mt, h_tt, e_su, e_du, e_sm, e_dm, e_st, e_dt,
               a_ut, a_mt, a_tt,
               sstage, dstage, pbig, rows4, sidx4, didx4, acc,
               gs0, gs1, gs2, gs3, gs4, gs5,
               ss0, ss1, ss2, ss3, ss4, ss5):
    core = lax.axis_index("c")
    sub = lax.axis_index("s")
    iota = lax.iota(jnp.int32, 16)
    gsems = [gs0, gs1, gs2, gs3, gs4, gs5]
    ssems = [ss0, ss1, ss2, ss3, ss4, ss5]
    NR = 6

    def zero_slot0():
        z = jnp.zeros((16,), jnp.float32)

        def body(i, _):
            for k in range(8):
                rows4[0, i, pl.ds(k * 16, 16)] = z
            return 0
        lax.fori_loop(0, 64, body, 0)

    for h_tab, e_src, e_dst, agg in ((h_ut, e_su, e_du, a_ut),
                                     (h_mt, e_sm, e_dm, a_mt),
                                     (h_tt, e_st, e_dt, a_tt)):

        def bucket_body(b, _, h_tab=h_tab, e_src=e_src, e_dst=e_dst,
                        agg=agg):
            lo = core * (4 * BUCK) + b * BUCK
            zero_slot0()
            for j in range(6):
                pltpu.sync_copy(rows4.at[0],
                                acc.at[pl.ds(sub * 400 + j * 64, 64), :])
            pltpu.sync_copy(rows4.at[0, pl.ds(0, 16), :],
                            acc.at[pl.ds(sub * 400 + 384, 16), :])
            plsc.subcore_barrier()

            lane15 = jnp.full((16,), 15, jnp.int32)

            def chunk_pass(c, cntv):
                pltpu.sync_copy(
                    e_src.at[pl.ds(sub * 112 + c * 56, 56), :], sstage)
                pltpu.sync_copy(
                    e_dst.at[pl.ds(sub * 112 + c * 56, 56), :], dstage)

                def row_body(r, cntv):
                    for k in range(8):
                        s16 = sstage[r, pl.ds(k * 16, 16)]
                        d16 = dstage[r, pl.ds(k * 16, 16)]
                        m = (d16 >= lo) & (d16 < lo + BUCK)
                        mi = jnp.where(m, 1, 0)
                        pos = cntv + plsc.cumsum(mi) - 1
                        pk = s16 | ((d16 - lo) << 17)
                        plsc.store_scatter(pbig, [pos >> 7, pos & 127], pk,
                                           mask=m)
                        cntv = _take16(pos, lane15) + 1
                    return cntv
                return lax.fori_loop(0, 56, row_body, cntv)

            cntv = lax.fori_loop(0, 2, chunk_pass,
                                 jnp.zeros((16,), jnp.int32))
            didx4[0, pl.ds(0, 16)] = cntv
            cnt = didx4[0, pl.ds(0, 16)][0]
            nb = (cnt + 63) >> 6
            hi = nb << 6
            pdump = jnp.full((16,), DUMP << 17, jnp.int32)
            for k in range(4):
                p = cnt + k * 16 + iota
                m = p < hi
                plsc.store_scatter(pbig, [p >> 7, p & 127], pdump, mask=m)

            def super_body(sup, _):
                base = sup * NR
                for q in range(NR):
                    j = base + q

                    @pl.when((j < nb) & (j >= NR))
                    def _():
                        pltpu.make_async_copy(rows4.at[q],
                                              acc.at[didx4.at[q]],
                                              ssems[q]).wait()

                    @pl.when(j < nb)
                    def _():
                        for k in range(4):
                            v = pbig[j >> 1, pl.ds((j & 1) * 64 + k * 16,
                                                   16)]
                            sidx4[q, pl.ds(k * 16, 16)] = v & 131071
                            didx4[q, pl.ds(k * 16, 16)] = v >> 17
                        pltpu.async_copy(h_tab.at[sidx4.at[q]], rows4.at[q],
                                         gsems[q])
                for q in range(NR):
                    j = base + q

                    @pl.when(j < nb)
                    def _():
                        pltpu.make_async_copy(h_tab.at[sidx4.at[q]],
                                              rows4.at[q], gsems[q]).wait()
                        pltpu.async_copy(rows4.at[q], acc.at[didx4.at[q]],
                                         ssems[q], add=True)
                return 0

            nsup = (nb + NR - 1) // NR
            lax.fori_loop(0, nsup, super_body, 0)
            for q in range(NR):

                @pl.when(q < nb)
                def _():
                    pltpu.make_async_copy(rows4.at[q], acc.at[didx4.at[q]],
                                          ssems[q]).wait()
            plsc.subcore_barrier()

            @pl.when(sub < 15)
            def _():
                pltpu.sync_copy(acc.at[pl.ds(sub * 392, 392), :],
                                agg.at[pl.ds(lo + sub * 392, 392), :])

            @pl.when(sub == 15)
            def _():
                pltpu.sync_copy(acc.at[pl.ds(5880, 376), :],
                                agg.at[pl.ds(lo + 5880, 376), :])
            plsc.subcore_barrier()
            return 0

        lax.fori_loop(0, 4, bucket_body, 0)


@functools.partial(
    pl.kernel, mesh=_mesh,
    compiler_params=pltpu.CompilerParams(needs_layout_passes=False),
    out_type=[jax.ShapeDtypeStruct((AGG_R, D), jnp.float32),
              jax.ShapeDtypeStruct((AGG_R, D), jnp.float32),
              jax.ShapeDtypeStruct((AGG_R, D), jnp.float32)],
    scratch_types=[pltpu.VMEM((56, 128), jnp.int32),
                   pltpu.VMEM((56, 128), jnp.int32),
                   pltpu.VMEM((112, 128), jnp.int32),
                   pltpu.VMEM((6, 64, 128), jnp.float32),
                   pltpu.VMEM((6, 64), jnp.int32),
                   pltpu.VMEM((6, 64), jnp.int32),
                   pltpu.VMEM_SHARED((ACC_R, 128), jnp.float32),
                   pltpu.SemaphoreType.DMA, pltpu.SemaphoreType.DMA,
                   pltpu.SemaphoreType.DMA, pltpu.SemaphoreType.DMA,
                   pltpu.SemaphoreType.DMA, pltpu.SemaphoreType.DMA,
                   pltpu.SemaphoreType.DMA, pltpu.SemaphoreType.DMA,
                   pltpu.SemaphoreType.DMA, pltpu.SemaphoreType.DMA,
                   pltpu.SemaphoreType.DMA, pltpu.SemaphoreType.DMA])
def _conv_kernel(*args):
    _conv_body(*args)


def _head_body(ag_ut, ag_mt, ag_tt, id_ut, id_mt, id_tt,
               w_ut, w_mt, w_tt, bsum, w_out, b_out,
               emb_ref, logits_ref):
    def rs(d_ref):
        d = d_ref[0, :] + d_ref[1, :]
        return lax.rsqrt(jnp.maximum(d, 1.0))[:, None]

    e = jnp.dot(ag_ut[...] * rs(id_ut), w_ut[...],
                preferred_element_type=jnp.float32)
    e += jnp.dot(ag_mt[...] * rs(id_mt), w_mt[...],
                 preferred_element_type=jnp.float32)
    e += jnp.dot(ag_tt[...] * rs(id_tt), w_tt[...],
                 preferred_element_type=jnp.float32)
    e += bsum[...]
    emb_ref[...] = e
    logits_ref[...] = jnp.dot(e, w_out[...],
                              preferred_element_type=jnp.float32) + b_out[...]


def _head(ag_ut, ag_mt, ag_tt, in_ut, in_mt, in_tt,
          W_ut, W_mt, W_tt, bsum, W_out, b_out):
    blk = 512
    nb = pl.cdiv(N_T, blk)
    row = pl.BlockSpec((blk, D), lambda i: (i, 0))
    col = pl.BlockSpec((2, blk), lambda i: (0, i))
    full = pl.BlockSpec((D, D), lambda i: (0, 0))
    return pl.pallas_call(
        _head_body,
        grid=(nb,),
        in_specs=[row, row, row, col, col, col, full, full, full,
                  pl.BlockSpec((1, D), lambda i: (0, 0)),
                  pl.BlockSpec((D, 1), lambda i: (0, 0)),
                  pl.BlockSpec((1, 1), lambda i: (0, 0))],
        out_specs=[row, pl.BlockSpec((blk, 1), lambda i: (i, 0))],
        out_shape=[jax.ShapeDtypeStruct((N_T, D), jnp.float32),
                   jax.ShapeDtypeStruct((N_T, 1), jnp.float32)],
    )(ag_ut, ag_mt, ag_tt, in_ut, in_mt, in_tt,
      W_ut, W_mt, W_tt, bsum, W_out, b_out)


def kernel(features, user_ids, merchant_ids,
           src_ut, dst_ut, src_mt, dst_mt, src_tt, dst_tt,
           emb_user, emb_merchant,
           W_ut, b_ut, W_mt, b_mt, W_tt, b_tt, W_out, b_out):
    h_user = jnp.take(emb_user, user_ids, axis=0)
    h_merchant = jnp.take(emb_merchant, merchant_ids, axis=0)

    e_su = _pad_edges(src_ut, N_U)
    e_du = _pad_edges(dst_ut, DPAD)
    e_sm = _pad_edges(src_mt, N_M)
    e_dm = _pad_edges(dst_mt, DPAD)
    e_st = _pad_edges(src_tt, N_T)
    e_dt = _pad_edges(dst_tt, DPAD)

    o_ut, o_mt, o_tt, i_ut, i_mt, i_tt = _hist_kernel(
        e_su, e_du, e_sm, e_dm, e_st, e_dt)

    hn_ut = _norm(h_user, o_ut.reshape(2, LU), N_U)
    hn_mt = _norm(h_merchant, o_mt.reshape(2, LM), N_M)
    hn_tt = _norm(features, o_tt.reshape(2, LT), N_T)

    ag_ut, ag_mt, ag_tt = _conv_kernel(
        hn_ut, hn_mt, hn_tt, e_su, e_du, e_sm, e_dm, e_st, e_dt)

    bsum = (b_ut + b_mt + b_tt).reshape(1, D)
    emb, logits = _head(ag_ut[:N_T], ag_mt[:N_T], ag_tt[:N_T],
                        i_ut.reshape(2, LT), i_mt.reshape(2, LT),
                        i_tt.reshape(2, LT),
                        W_ut, W_mt, W_tt, bsum, W_out, b_out.reshape(1, 1))
    return (logits, emb)


# speedup vs baseline: 1.5657x; 1.1865x over previous
import functools

import jax
import jax.numpy as jnp
from jax import lax
from jax.experimental import pallas as pl
from jax.experimental.pallas import tpu as pltpu
from jax.experimental.pallas import tpu_sc as plsc

N_T = 50000
N_U = 10000
N_M = 5000
D = 128
E = 200000

ER = 1792
EP = ER * 128
BUCK = 6256
AGG_R = 8 * BUCK
ACC_R = 6400
DUMP = BUCK + 8
DPAD = 50112
FLUSH = 112

LU = 10240
LM = 6144
LT = 51200

_mesh = plsc.VectorSubcoreMesh(core_axis_name="c", subcore_axis_name="s")

_GDN = lax.GatherDimensionNumbers(offset_dims=(), collapsed_slice_dims=(0,),
                                  start_index_map=(0,))


def _take16(vec, idx):
    return lax.gather(vec, idx[:, None], _GDN, (1,),
                      mode=lax.GatherScatterMode.PROMISE_IN_BOUNDS)


def _pad_edges(a, fill):
    pad = jnp.full((EP - E,), fill, jnp.int32)
    return jnp.concatenate([a.astype(jnp.int32), pad]).reshape(ER, 128)


def _hist_body(s_ut, d_ut, s_mt, d_mt, s_tt, d_tt,
               o_ut, o_mt, o_tt, i_ut, i_mt, i_tt,
               stage, ones_v, zbuf,
               h_ou, h_om, h_ot, h_iu, h_im, h_it, hsem):
    core = lax.axis_index("c")
    sub = lax.axis_index("s")

    def fill(ref, n, val):
        v = jnp.full((16,), val, jnp.float32)

        def body(i, _):
            ref[pl.ds(i * 16, 16)] = v
            return 0
        lax.fori_loop(0, n // 16, body, 0)

    fill(zbuf, LT // 16, 0.0)
    fill(ones_v, 128, 1.0)

    pairs = [(s_ut, h_ou, o_ut, LU), (s_mt, h_om, o_mt, LM),
             (s_tt, h_ot, o_tt, LT), (d_ut, h_iu, i_ut, LT),
             (d_mt, h_im, i_mt, LT), (d_tt, h_it, i_tt, LT)]

    for _, hist, _, L in pairs:
        share = L // 16
        pltpu.sync_copy(zbuf.at[pl.ds(0, share)],
                        hist.at[pl.ds(sub * share, share)])
    plsc.subcore_barrier()

    row0 = core * (ER // 2) + sub * 56
    pltpu.sync_copy(pairs[0][0].at[pl.ds(row0, 56), :], stage.at[0])
    for k, (arr, hist, _, L) in enumerate(pairs):
        sl = k & 1
        for g in range(56):
            pltpu.async_copy(ones_v, hist.at[stage.at[sl, g]], hsem,
                             add=True)
        if k + 1 < len(pairs):
            pltpu.sync_copy(pairs[k + 1][0].at[pl.ds(row0, 56), :],
                            stage.at[1 - sl])
        for g in range(56):
            pltpu.make_async_copy(ones_v, hist.at[stage.at[sl, g]],
                                  hsem).wait()
    plsc.subcore_barrier()

    for _, hist, out, L in pairs:
        share = L // 16
        pltpu.sync_copy(hist.at[pl.ds(sub * share, share)],
                        out.at[pl.ds(core * L + sub * share, share)])


@functools.partial(
    pl.kernel, mesh=_mesh,
    compiler_params=pltpu.CompilerParams(needs_layout_passes=False),
    out_type=[jax.ShapeDtypeStruct((2 * LU,), jnp.float32),
              jax.ShapeDtypeStruct((2 * LM,), jnp.float32),
              jax.ShapeDtypeStruct((2 * LT,), jnp.float32),
              jax.ShapeDtypeStruct((2 * LT,), jnp.float32),
              jax.ShapeDtypeStruct((2 * LT,), jnp.float32),
              jax.ShapeDtypeStruct((2 * LT,), jnp.float32)],
    scratch_types=[pltpu.VMEM((2, 56, 128), jnp.int32),
                   pltpu.VMEM((128,), jnp.float32),
                   pltpu.VMEM((LT // 16,), jnp.float32),
                   pltpu.VMEM_SHARED((LU,), jnp.float32),
                   pltpu.VMEM_SHARED((LM,), jnp.float32),
                   pltpu.VMEM_SHARED((LT,), jnp.float32),
                   pltpu.VMEM_SHARED((LT,), jnp.float32),
                   pltpu.VMEM_SHARED((LT,), jnp.float32),
                   pltpu.VMEM_SHARED((LT,), jnp.float32),
                   pltpu.SemaphoreType.DMA])
def _hist_kernel(*args):
    _hist_body(*args)


def _norm_body(t_ref, d_ref, o_ref):
    d = d_ref[0, :] + d_ref[1, :]
    o_ref[...] = t_ref[...] * lax.rsqrt(jnp.maximum(d, 1.0))[:, None]


def _norm(table, deg, n):
    blk = 512
    nb = pl.cdiv(n, blk)
    return pl.pallas_call(
        _norm_body,
        grid=(nb,),
        in_specs=[pl.BlockSpec((blk, D), lambda i: (i, 0)),
                  pl.BlockSpec((2, blk), lambda i: (0, i))],
        out_specs=pl.BlockSpec((blk, D), lambda i: (i, 0)),
        out_shape=jax.ShapeDtypeStruct((n, D), jnp.float32),
    )(table, deg)


def _conv_body(h_ut, h_mt, h_tt, e_su, e_du, e_sm, e_dm, e_st, e_dt,
               a_ut, a_mt, a_tt,
               sstage, dstage, pbig, rows4, sidx4, didx4, acc,
               gs0, gs1, gs2, gs3, gs4, gs5, gs6, gs7, gs8, gs9,
               ss0, ss1, ss2, ss3, ss4, ss5, ss6, ss7, ss8, ss9):
    core = lax.axis_index("c")
    sub = lax.axis_index("s")
    iota = lax.iota(jnp.int32, 16)
    gsems = [gs0, gs1, gs2, gs3, gs4, gs5, gs6, gs7, gs8, gs9]
    ssems = [ss0, ss1, ss2, ss3, ss4, ss5, ss6, ss7, ss8, ss9]
    NR = 10

    def zero_slot0():
        z = jnp.zeros((16,), jnp.float32)

        def body(i, _):
            for k in range(8):
                rows4[0, i, pl.ds(k * 16, 16)] = z
            return 0
        lax.fori_loop(0, 32, body, 0)

    for h_tab, e_src, e_dst, agg in ((h_ut, e_su, e_du, a_ut),
                                     (h_mt, e_sm, e_dm, a_mt),
                                     (h_tt, e_st, e_dt, a_tt)):

        def bucket_body(b, _, h_tab=h_tab, e_src=e_src, e_dst=e_dst,
                        agg=agg):
            lo = core * (4 * BUCK) + b * BUCK
            zero_slot0()
            for j in range(12):
                pltpu.sync_copy(rows4.at[0],
                                acc.at[pl.ds(sub * 400 + j * 32, 32), :])
            pltpu.sync_copy(rows4.at[0, pl.ds(0, 16), :],
                            acc.at[pl.ds(sub * 400 + 384, 16), :])
            plsc.subcore_barrier()

            lane15 = jnp.full((16,), 15, jnp.int32)

            def chunk_pass(c, cntv):
                pltpu.sync_copy(
                    e_src.at[pl.ds(sub * 112 + c * 56, 56), :], sstage)
                pltpu.sync_copy(
                    e_dst.at[pl.ds(sub * 112 + c * 56, 56), :], dstage)

                def row_body(r, cntv):
                    for k in range(8):
                        s16 = sstage[r, pl.ds(k * 16, 16)]
                        d16 = dstage[r, pl.ds(k * 16, 16)]
                        m = (d16 >= lo) & (d16 < lo + BUCK)
                        mi = jnp.where(m, 1, 0)
                        pos = cntv + plsc.cumsum(mi) - 1
                        pk = s16 | ((d16 - lo) << 17)
                        plsc.store_scatter(pbig, [pos >> 7, pos & 127], pk,
                                           mask=m)
                        cntv = _take16(pos, lane15) + 1
                    return cntv
                return lax.fori_loop(0, 56, row_body, cntv)

            cntv = lax.fori_loop(0, 2, chunk_pass,
                                 jnp.zeros((16,), jnp.int32))
            didx4[0, pl.ds(0, 16)] = cntv
            cnt = didx4[0, pl.ds(0, 16)][0]
            nb = (cnt + 31) >> 5
            hi = nb << 5
            pdump = jnp.full((16,), DUMP << 17, jnp.int32)
            for k in range(2):
                p = cnt + k * 16 + iota
                m = p < hi
                plsc.store_scatter(pbig, [p >> 7, p & 127], pdump, mask=m)

            def super_body(sup, _):
                base = sup * NR
                for q in range(NR):
                    j = base + q

                    @pl.when((j < nb) & (j >= NR))
                    def _():
                        pltpu.make_async_copy(rows4.at[q],
                                              acc.at[didx4.at[q]],
                                              ssems[q]).wait()

                    @pl.when(j < nb)
                    def _():
                        for k in range(2):
                            v = pbig[j >> 2, pl.ds((j & 3) * 32 + k * 16,
                                                   16)]
                            sidx4[q, pl.ds(k * 16, 16)] = v & 131071
                            didx4[q, pl.ds(k * 16, 16)] = v >> 17
                        pltpu.async_copy(h_tab.at[sidx4.at[q]], rows4.at[q],
                                         gsems[q])
                for q in range(NR):
                    j = base + q

                    @pl.when(j < nb)
                    def _():
                        pltpu.make_async_copy(h_tab.at[sidx4.at[q]],
                                              rows4.at[q], gsems[q]).wait()
                        pltpu.async_copy(rows4.at[q], acc.at[didx4.at[q]],
                                         ssems[q], add=True)
                return 0

            nsup = (nb + NR - 1) // NR
            lax.fori_loop(0, nsup, super_body, 0)
            for q in range(NR):

                @pl.when(q < nb)
                def _():
                    pltpu.make_async_copy(rows4.at[q], acc.at[didx4.at[q]],
                                          ssems[q]).wait()
            plsc.subcore_barrier()

            @pl.when(sub < 15)
            def _():
                pltpu.sync_copy(acc.at[pl.ds(sub * 392, 392), :],
                                agg.at[pl.ds(lo + sub * 392, 392), :])

            @pl.when(sub == 15)
            def _():
                pltpu.sync_copy(acc.at[pl.ds(5880, 376), :],
                                agg.at[pl.ds(lo + 5880, 376), :])
            plsc.subcore_barrier()
            return 0

        lax.fori_loop(0, 4, bucket_body, 0)


@functools.partial(
    pl.kernel, mesh=_mesh,
    compiler_params=pltpu.CompilerParams(needs_layout_passes=False),
    out_type=[jax.ShapeDtypeStruct((AGG_R, D), jnp.float32),
              jax.ShapeDtypeStruct((AGG_R, D), jnp.float32),
              jax.ShapeDtypeStruct((AGG_R, D), jnp.float32)],
    scratch_types=[pltpu.VMEM((56, 128), jnp.int32),
                   pltpu.VMEM((56, 128), jnp.int32),
                   pltpu.VMEM((112, 128), jnp.int32),
                   pltpu.VMEM((10, 32, 128), jnp.float32),
                   pltpu.VMEM((10, 32), jnp.int32),
                   pltpu.VMEM((10, 32), jnp.int32),
                   pltpu.VMEM_SHARED((ACC_R, 128), jnp.float32)]
                  + [pltpu.SemaphoreType.DMA] * 20)
def _conv_kernel(*args):
    _conv_body(*args)


def _head_body(ag_ut, ag_mt, ag_tt, id_ut, id_mt, id_tt,
               w_ut, w_mt, w_tt, bsum, w_out, b_out,
               emb_ref, logits_ref):
    def rs(d_ref):
        d = d_ref[0, :] + d_ref[1, :]
        return lax.rsqrt(jnp.maximum(d, 1.0))[:, None]

    e = jnp.dot(ag_ut[...] * rs(id_ut), w_ut[...],
                preferred_element_type=jnp.float32)
    e += jnp.dot(ag_mt[...] * rs(id_mt), w_mt[...],
                 preferred_element_type=jnp.float32)
    e += jnp.dot(ag_tt[...] * rs(id_tt), w_tt[...],
                 preferred_element_type=jnp.float32)
    e += bsum[...]
    emb_ref[...] = e
    logits_ref[...] = jnp.dot(e, w_out[...],
                              preferred_element_type=jnp.float32) + b_out[...]


def _head(ag_ut, ag_mt, ag_tt, in_ut, in_mt, in_tt,
          W_ut, W_mt, W_tt, bsum, W_out, b_out):
    blk = 512
    nb = pl.cdiv(N_T, blk)
    row = pl.BlockSpec((blk, D), lambda i: (i, 0))
    col = pl.BlockSpec((2, blk), lambda i: (0, i))
    full = pl.BlockSpec((D, D), lambda i: (0, 0))
    return pl.pallas_call(
        _head_body,
        grid=(nb,),
        in_specs=[row, row, row, col, col, col, full, full, full,
                  pl.BlockSpec((1, D), lambda i: (0, 0)),
                  pl.BlockSpec((D, 1), lambda i: (0, 0)),
                  pl.BlockSpec((1, 1), lambda i: (0, 0))],
        out_specs=[row, pl.BlockSpec((blk, 1), lambda i: (i, 0))],
        out_shape=[jax.ShapeDtypeStruct((N_T, D), jnp.float32),
                   jax.ShapeDtypeStruct((N_T, 1), jnp.float32)],
    )(ag_ut, ag_mt, ag_tt, in_ut, in_mt, in_tt,
      W_ut, W_mt, W_tt, bsum, W_out, b_out)


def kernel(features, user_ids, merchant_ids,
           src_ut, dst_ut, src_mt, dst_mt, src_tt, dst_tt,
           emb_user, emb_merchant,
           W_ut, b_ut, W_mt, b_mt, W_tt, b_tt, W_out, b_out):
    h_user = jnp.take(emb_user, user_ids, axis=0)
    h_merchant = jnp.take(emb_merchant, merchant_ids, axis=0)

    e_su = _pad_edges(src_ut, N_U)
    e_du = _pad_edges(dst_ut, DPAD)
    e_sm = _pad_edges(src_mt, N_M)
    e_dm = _pad_edges(dst_mt, DPAD)
    e_st = _pad_edges(src_tt, N_T)
    e_dt = _pad_edges(dst_tt, DPAD)

    o_ut, o_mt, o_tt, i_ut, i_mt, i_tt = _hist_kernel(
        e_su, e_du, e_sm, e_dm, e_st, e_dt)

    hn_ut = _norm(h_user, o_ut.reshape(2, LU), N_U)
    hn_mt = _norm(h_merchant, o_mt.reshape(2, LM), N_M)
    hn_tt = _norm(features, o_tt.reshape(2, LT), N_T)

    ag_ut, ag_mt, ag_tt = _conv_kernel(
        hn_ut, hn_mt, hn_tt, e_su, e_du, e_sm, e_dm, e_st, e_dt)

    bsum = (b_ut + b_mt + b_tt).reshape(1, D)
    emb, logits = _head(ag_ut[:N_T], ag_mt[:N_T], ag_tt[:N_T],
                        i_ut.reshape(2, LT), i_mt.reshape(2, LT),
                        i_tt.reshape(2, LT),
                        W_ut, W_mt, W_tt, bsum, W_out, b_out.reshape(1, 1))
    return (logits, emb)
